# baseline (speedup 1.0000x reference)
import functools

import jax
import jax.numpy as jnp
from jax import lax
from jax.experimental import pallas as pl
from jax.experimental.pallas import tpu as pltpu
from jax.experimental.pallas import tpu_sc as plsc

VOCAB = 1000000
EMBED_DIM = 64
EMBED_PAD = 128
BATCH = 4096
SEQ = 200
B_TOTAL = BATCH * SEQ

NUM_CORES = 2
NUM_SUBCORES = 16
NUM_WORKERS = NUM_CORES * NUM_SUBCORES
B_PER_W = B_TOTAL // NUM_WORKERS

NBUF = 2
CHUNK = 200
GROUP = NBUF * CHUNK
N_GROUPS = B_PER_W // GROUP
assert B_PER_W % GROUP == 0

LANES = 16
ROW_UNROLL = 8
assert CHUNK % ROW_UNROLL == 0


def _make_emb_kernel():
    mesh = plsc.VectorSubcoreMesh(core_axis_name="c", subcore_axis_name="s")

    scratch = (
        [pltpu.VMEM((CHUNK,), jnp.int32) for _ in range(NBUF)]
        + [pltpu.VMEM((CHUNK, EMBED_PAD), jnp.float32) for _ in range(NBUF)]
        + [pltpu.VMEM((CHUNK, EMBED_DIM), jnp.float32) for _ in range(NBUF)]
        + [pltpu.SemaphoreType.DMA for _ in range(3 * NBUF)]
    )

    @functools.partial(
        pl.kernel,
        mesh=mesh,
        out_type=jax.ShapeDtypeStruct((B_TOTAL, EMBED_DIM), jnp.float32),
        scratch_types=scratch,
    )
    def emb_kernel(idx_hbm, table_hbm, out_hbm, *scr):
        idx_vs = scr[:NBUF]
        rows_vs = scr[NBUF : 2 * NBUF]
        cmp_vs = scr[2 * NBUF : 3 * NBUF]
        idx_sems = scr[3 * NBUF : 4 * NBUF]
        gat_sems = scr[4 * NBUF : 5 * NBUF]
        out_sems = scr[5 * NBUF : 6 * NBUF]

        wid = lax.axis_index("s") * NUM_CORES + lax.axis_index("c")
        base0 = wid * B_PER_W

        for b in range(NBUF):
            pltpu.async_copy(
                idx_hbm.at[pl.ds(base0 + b * CHUNK, CHUNK)], idx_vs[b], idx_sems[b]
            )

        def group_body(g, carry):
            base_g = base0 + g * GROUP
            for b in range(NBUF):
                pltpu.make_async_copy(
                    idx_hbm.at[pl.ds(base_g + b * CHUNK, CHUNK)],
                    idx_vs[b],
                    idx_sems[b],
                ).wait()
                pltpu.async_copy(
                    table_hbm.at[idx_vs[b]], rows_vs[b], gat_sems[b]
                )
            for b in range(NBUF):
                chunk_base = base_g + b * CHUNK
                pltpu.make_async_copy(
                    table_hbm.at[idx_vs[b]], rows_vs[b], gat_sems[b]
                ).wait()

                rows = rows_vs[b]
                cmp = cmp_vs[b]

                def compact(i, c, rows=rows, cmp=cmp):
                    for u in range(ROW_UNROLL):
                        r = i * ROW_UNROLL + u
                        for v in range(EMBED_DIM // LANES):
                            cmp[r, pl.ds(v * LANES, LANES)] = rows[
                                r, pl.ds(v * LANES, LANES)
                            ]
                    return c

                lax.fori_loop(0, CHUNK // ROW_UNROLL, compact, 0)

                pltpu.async_copy(
                    cmp, out_hbm.at[pl.ds(chunk_base, CHUNK)], out_sems[b]
                )

                @pl.when(g + 1 < N_GROUPS)
                def _prefetch(b=b, base_g=base_g):
                    pltpu.async_copy(
                        idx_hbm.at[pl.ds(base_g + GROUP + b * CHUNK, CHUNK)],
                        idx_vs[b],
                        idx_sems[b],
                    )

            for b in range(NBUF):
                pltpu.make_async_copy(
                    cmp_vs[b],
                    out_hbm.at[pl.ds(base_g + b * CHUNK, CHUNK)],
                    out_sems[b],
                ).wait()
            return carry

        lax.fori_loop(0, N_GROUPS, group_body, 0)

    return emb_kernel


_emb = _make_emb_kernel()


def kernel(x, table):
    idx = x.reshape(-1).astype(jnp.int32)
    table_pad = jnp.pad(table, ((0, 0), (0, EMBED_PAD - EMBED_DIM)))
    out = _emb(idx, table_pad)
    return out.reshape(BATCH, SEQ, EMBED_DIM)
